# Initial kernel scaffold; baseline (speedup 1.0000x reference)
import functools

import jax
import jax.numpy as jnp
from jax import lax
from jax.experimental import pallas as pl
from jax.experimental.pallas import tpu as pltpu
from jax.experimental.pallas import tpu_sc as plsc

N = 10000
E = 320000
H = 128
L = 6
T = 128
G = 64

NC = 2
NS = 16
NW = NC * NS
EPT = E // NW
CH = 80
NCHUNK = EPT // CH
RPT = N // NS

BN = 1000

_mesh = plsc.VectorSubcoreMesh(core_axis_name="c", subcore_axis_name="s")


@functools.partial(
    pl.kernel,
    out_type=jax.ShapeDtypeStruct((NC, N, H), jnp.float32),
    mesh=_mesh,
    scratch_types=[
        pltpu.VMEM((NCHUNK, CH), jnp.int32),
        pltpu.VMEM((NCHUNK, CH), jnp.int32),
        pltpu.VMEM((CH, H), jnp.float32),
        pltpu.VMEM_SHARED((N, H), jnp.float32),
        pltpu.SemaphoreType.DMA,
    ],
)
def _sc_agg(h_hbm, src_hbm, dst_hbm, out_hbm, srcv, dstv, rows, slab, sem):
    c = lax.axis_index("c")
    s = lax.axis_index("s")
    wid = s * NC + c
    pltpu.sync_copy(h_hbm.at[pl.ds(s * RPT, RPT)], slab.at[pl.ds(s * RPT, RPT)])
    pltpu.sync_copy(src_hbm.at[wid], srcv)
    pltpu.sync_copy(dst_hbm.at[wid], dstv)
    plsc.subcore_barrier()

    @pl.loop(0, NCHUNK)
    def _edges(j):
        pltpu.async_copy(h_hbm.at[srcv.at[j]], rows, sem).wait()
        pltpu.sync_copy(rows, slab.at[dstv.at[j]], add=True)

    plsc.subcore_barrier()
    pltpu.sync_copy(slab.at[pl.ds(s * RPT, RPT)],
                    out_hbm.at[c].at[pl.ds(s * RPT, RPT)])


def _embed_body(x_ref, emb_ref, o_ref):
    xv = x_ref[...]
    onehot = (xv == lax.broadcasted_iota(jnp.int32, (BN, T), 1)).astype(jnp.float32)
    o_ref[...] = jnp.dot(onehot, emb_ref[...], preferred_element_type=jnp.float32)


def _embed(x2, emb):
    return pl.pallas_call(
        _embed_body,
        grid=(N // BN,),
        in_specs=[
            pl.BlockSpec((BN, 1), lambda i: (i, 0)),
            pl.BlockSpec((T, H), lambda i: (0, 0)),
        ],
        out_specs=pl.BlockSpec((BN, H), lambda i: (i, 0)),
        out_shape=jax.ShapeDtypeStruct((N, H), jnp.float32),
    )(x2, emb)


def _mlp_body(h_ref, a0_ref, a1_ref, w1_ref, b1_ref, w2_ref, b2_ref, o_ref):
    z = a0_ref[...] + a1_ref[...] - h_ref[...]
    y = lax.dot_general(z, w1_ref[...], (((1,), (1,)), ((), ())),
                        preferred_element_type=jnp.float32) + b1_ref[...]
    y = jnp.maximum(y, 0.0)
    y = lax.dot_general(y, w2_ref[...], (((1,), (1,)), ((), ())),
                        preferred_element_type=jnp.float32) + b2_ref[...]
    o_ref[...] = jnp.maximum(y, 0.0)


def _mlp(h, a0, a1, w1, b1, w2, b2):
    full = lambda i: (0, 0)
    return pl.pallas_call(
        _mlp_body,
        grid=(N // BN,),
        in_specs=[
            pl.BlockSpec((BN, H), lambda i: (i, 0)),
            pl.BlockSpec((BN, H), lambda i: (i, 0)),
            pl.BlockSpec((BN, H), lambda i: (i, 0)),
            pl.BlockSpec((H, H), full),
            pl.BlockSpec((1, H), full),
            pl.BlockSpec((H, H), full),
            pl.BlockSpec((1, H), full),
        ],
        out_specs=pl.BlockSpec((BN, H), lambda i: (i, 0)),
        out_shape=jax.ShapeDtypeStruct((N, H), jnp.float32),
    )(h, a0, a1, w1, b1, w2, b2)


def _pool_body(b_ref, h_ref, o_ref, cnt_ref):
    i = pl.program_id(0)

    @pl.when(i == 0)
    def _():
        o_ref[...] = jnp.zeros_like(o_ref)
        cnt_ref[...] = jnp.zeros_like(cnt_ref)

    bv = b_ref[...]
    onehot = (bv == lax.broadcasted_iota(jnp.int32, (BN, G), 1)).astype(jnp.float32)
    o_ref[...] += lax.dot_general(onehot, h_ref[...], (((0,), (0,)), ((), ())),
                                  preferred_element_type=jnp.float32)
    cnt_ref[...] += lax.dot_general(onehot, jnp.ones((BN, H), jnp.float32),
                                    (((0,), (0,)), ((), ())),
                                    preferred_element_type=jnp.float32)

    @pl.when(i == pl.num_programs(0) - 1)
    def _():
        o_ref[...] = o_ref[...] / jnp.maximum(cnt_ref[...], 1.0)


def _pool(b2d, h):
    return pl.pallas_call(
        _pool_body,
        grid=(N // BN,),
        in_specs=[
            pl.BlockSpec((BN, 1), lambda i: (i, 0)),
            pl.BlockSpec((BN, H), lambda i: (i, 0)),
        ],
        out_specs=pl.BlockSpec((G, H), lambda i: (0, 0)),
        out_shape=jax.ShapeDtypeStruct((G, H), jnp.float32),
        scratch_shapes=[pltpu.VMEM((G, H), jnp.float32)],
    )(b2d, h)


def kernel(x, edge_index, batch, emb, W1, b1, W2, b2):
    x2 = x.astype(jnp.int32).reshape(N, 1)
    b2d = batch.astype(jnp.int32).reshape(N, 1)
    src = edge_index[0].astype(jnp.int32).reshape(NW, NCHUNK, CH)
    dst = edge_index[1].astype(jnp.int32).reshape(NW, NCHUNK, CH)

    h = _embed(x2, emb)
    for l in range(L):
        agg = _sc_agg(h, src, dst)
        h = _mlp(h, agg[0], agg[1], W1[l], b1[l].reshape(1, H),
                 W2[l], b2[l].reshape(1, H))
    return _pool(b2d, h)


# speedup vs baseline: 6.8621x; 6.8621x over previous
import functools

import jax
import jax.numpy as jnp
from jax import lax
from jax.experimental import pallas as pl
from jax.experimental.pallas import tpu as pltpu
from jax.experimental.pallas import tpu_sc as plsc

N = 10000
E = 320000
H = 128
L = 6
T = 128
G = 64

NC = 2
NS = 16
NW = NC * NS
EPT = E // NW
CH = 80
NCHUNK = EPT // CH
RPT = 640
RPT_LAST = N - (NS - 1) * RPT

BN = 1000

@functools.cache
def _build_sc_agg():
    mesh = plsc.VectorSubcoreMesh(core_axis_name="c", subcore_axis_name="s",
                                  num_cores=NC, num_subcores=NS)

    @functools.partial(
        pl.kernel,
        out_type=jax.ShapeDtypeStruct((NC, N, H), jnp.float32),
        mesh=mesh,
        scratch_types=[
            pltpu.VMEM((NCHUNK, CH), jnp.int32),
            pltpu.VMEM((NCHUNK, CH), jnp.int32),
            pltpu.VMEM((CH, H), jnp.float32),
            pltpu.VMEM_SHARED((N, H), jnp.float32),
            pltpu.SemaphoreType.DMA,
        ],
    )
    def sc_agg(h_hbm, src_hbm, dst_hbm, out_hbm, srcv, dstv, rows, slab, sem):
        c = lax.axis_index("c")
        s = lax.axis_index("s")
        wid = s * NC + c
        @pl.when(s < NS - 1)
        def _():
            pltpu.sync_copy(h_hbm.at[pl.ds(s * RPT, RPT)],
                            slab.at[pl.ds(s * RPT, RPT)])

        @pl.when(s == NS - 1)
        def _():
            pltpu.sync_copy(h_hbm.at[pl.ds((NS - 1) * RPT, RPT_LAST)],
                            slab.at[pl.ds((NS - 1) * RPT, RPT_LAST)])

        pltpu.sync_copy(src_hbm.at[wid], srcv)
        pltpu.sync_copy(dst_hbm.at[wid], dstv)
        plsc.subcore_barrier()

        @pl.loop(0, NCHUNK)
        def _edges(j):
            pltpu.async_copy(h_hbm.at[srcv.at[j]], rows, sem).wait()
            pltpu.sync_copy(rows, slab.at[dstv.at[j]], add=True)

        plsc.subcore_barrier()

        @pl.when(s < NS - 1)
        def _():
            pltpu.sync_copy(slab.at[pl.ds(s * RPT, RPT)],
                            out_hbm.at[c].at[pl.ds(s * RPT, RPT)])

        @pl.when(s == NS - 1)
        def _():
            pltpu.sync_copy(slab.at[pl.ds((NS - 1) * RPT, RPT_LAST)],
                            out_hbm.at[c].at[pl.ds((NS - 1) * RPT, RPT_LAST)])

    return sc_agg


def _sc_agg(h, src, dst):
    return _build_sc_agg()(h, src, dst)


def _embed_body(x_ref, emb_ref, o_ref):
    xv = x_ref[...]
    onehot = (xv == lax.broadcasted_iota(jnp.int32, (BN, T), 1)).astype(jnp.float32)
    o_ref[...] = jnp.dot(onehot, emb_ref[...], preferred_element_type=jnp.float32)


def _embed(x2, emb):
    return pl.pallas_call(
        _embed_body,
        grid=(N // BN,),
        in_specs=[
            pl.BlockSpec((BN, 1), lambda i: (i, 0)),
            pl.BlockSpec((T, H), lambda i: (0, 0)),
        ],
        out_specs=pl.BlockSpec((BN, H), lambda i: (i, 0)),
        out_shape=jax.ShapeDtypeStruct((N, H), jnp.float32),
    )(x2, emb)


def _mlp_body(h_ref, a0_ref, a1_ref, w1_ref, b1_ref, w2_ref, b2_ref, o_ref):
    z = a0_ref[...] + a1_ref[...] - h_ref[...]
    y = lax.dot_general(z, w1_ref[...], (((1,), (1,)), ((), ())),
                        preferred_element_type=jnp.float32) + b1_ref[...]
    y = jnp.maximum(y, 0.0)
    y = lax.dot_general(y, w2_ref[...], (((1,), (1,)), ((), ())),
                        preferred_element_type=jnp.float32) + b2_ref[...]
    o_ref[...] = jnp.maximum(y, 0.0)


def _mlp(h, a0, a1, w1, b1, w2, b2):
    full = lambda i: (0, 0)
    return pl.pallas_call(
        _mlp_body,
        grid=(N // BN,),
        in_specs=[
            pl.BlockSpec((BN, H), lambda i: (i, 0)),
            pl.BlockSpec((BN, H), lambda i: (i, 0)),
            pl.BlockSpec((BN, H), lambda i: (i, 0)),
            pl.BlockSpec((H, H), full),
            pl.BlockSpec((1, H), full),
            pl.BlockSpec((H, H), full),
            pl.BlockSpec((1, H), full),
        ],
        out_specs=pl.BlockSpec((BN, H), lambda i: (i, 0)),
        out_shape=jax.ShapeDtypeStruct((N, H), jnp.float32),
    )(h, a0, a1, w1, b1, w2, b2)


def _pool_body(b_ref, h_ref, o_ref, cnt_ref):
    i = pl.program_id(0)

    @pl.when(i == 0)
    def _():
        o_ref[...] = jnp.zeros_like(o_ref)
        cnt_ref[...] = jnp.zeros_like(cnt_ref)

    bv = b_ref[...]
    onehot = (bv == lax.broadcasted_iota(jnp.int32, (BN, G), 1)).astype(jnp.float32)
    o_ref[...] += lax.dot_general(onehot, h_ref[...], (((0,), (0,)), ((), ())),
                                  preferred_element_type=jnp.float32)
    cnt_ref[...] += lax.dot_general(onehot, jnp.ones((BN, H), jnp.float32),
                                    (((0,), (0,)), ((), ())),
                                    preferred_element_type=jnp.float32)

    @pl.when(i == pl.num_programs(0) - 1)
    def _():
        o_ref[...] = o_ref[...] / jnp.maximum(cnt_ref[...], 1.0)


def _pool(b2d, h):
    return pl.pallas_call(
        _pool_body,
        grid=(N // BN,),
        in_specs=[
            pl.BlockSpec((BN, 1), lambda i: (i, 0)),
            pl.BlockSpec((BN, H), lambda i: (i, 0)),
        ],
        out_specs=pl.BlockSpec((G, H), lambda i: (0, 0)),
        out_shape=jax.ShapeDtypeStruct((G, H), jnp.float32),
        scratch_shapes=[pltpu.VMEM((G, H), jnp.float32)],
    )(b2d, h)


def kernel(x, edge_index, batch, emb, W1, b1, W2, b2):
    x2 = x.astype(jnp.int32).reshape(N, 1)
    b2d = batch.astype(jnp.int32).reshape(N, 1)
    src = edge_index[0].astype(jnp.int32).reshape(NW, NCHUNK, CH)
    dst = edge_index[1].astype(jnp.int32).reshape(NW, NCHUNK, CH)

    h = _embed(x2, emb)
    for l in range(L):
        agg = _sc_agg(h, src, dst)
        h = _mlp(h, agg[0], agg[1], W1[l], b1[l].reshape(1, H),
                 W2[l], b2[l].reshape(1, H))
    return _pool(b2d, h)


# speedup vs baseline: 11.0061x; 1.6039x over previous
import functools

import jax
import jax.numpy as jnp
from jax import lax
from jax.experimental import pallas as pl
from jax.experimental.pallas import tpu as pltpu
from jax.experimental.pallas import tpu_sc as plsc

N = 10000
E = 320000
H = 128
L = 6
T = 128
G = 64

NC = 2
NS = 16
NW = NC * NS
EPT = E // NW
CH = 80
NCHUNK = EPT // CH
RPT = 640
RPT_LAST = N - (NS - 1) * RPT

BN = 1000

@functools.cache
def _build_sc_agg():
    mesh = plsc.VectorSubcoreMesh(core_axis_name="c", subcore_axis_name="s",
                                  num_cores=NC, num_subcores=NS)

    @functools.partial(
        pl.kernel,
        out_type=jax.ShapeDtypeStruct((NC, N, H), jnp.float32),
        mesh=mesh,
        scratch_types=[
            pltpu.VMEM((EPT,), jnp.int32),
            pltpu.VMEM((NCHUNK, CH), jnp.int32),
            pltpu.VMEM((CH, H), jnp.float32),
            pltpu.VMEM((CH, H), jnp.float32),
            pltpu.VMEM_SHARED((N, H), jnp.float32),
            pltpu.SemaphoreType.DMA,
            pltpu.SemaphoreType.DMA,
        ],
    )
    def sc_agg(h_hbm, src_hbm, dst_hbm, out_hbm, srcv, dstv,
               rows0, rows1, slab, sem0, sem1):
        c = lax.axis_index("c")
        s = lax.axis_index("s")
        wid = s * NC + c
        @pl.when(s < NS - 1)
        def _():
            pltpu.sync_copy(h_hbm.at[pl.ds(s * RPT, RPT)],
                            slab.at[pl.ds(s * RPT, RPT)])

        @pl.when(s == NS - 1)
        def _():
            pltpu.sync_copy(h_hbm.at[pl.ds((NS - 1) * RPT, RPT_LAST)],
                            slab.at[pl.ds((NS - 1) * RPT, RPT_LAST)])

        pltpu.sync_copy(src_hbm.at[wid], srcv)
        pltpu.sync_copy(dst_hbm.at[wid], dstv)

        bufs = [(rows0, sem0), (rows1, sem1)]
        NB = len(bufs)
        for b in range(NB):
            pltpu.async_copy(h_hbm.at[srcv.at[pl.ds(b * CH, CH)]],
                             bufs[b][0], bufs[b][1])
        plsc.subcore_barrier()

        MAIN = (NCHUNK // NB) * NB

        @pl.loop(0, MAIN, step=NB)
        def _edges(j):
            for b in range(NB):
                rows_b, sem_b = bufs[b]
                jj = j + b
                pltpu.make_async_copy(h_hbm.at[srcv.at[pl.ds(jj * CH, CH)]],
                                      rows_b, sem_b).wait()
                pltpu.sync_copy(rows_b, slab.at[dstv.at[jj]], add=True)
                nxt = jj + NB

                @pl.when(nxt < NCHUNK)
                def _():
                    pltpu.async_copy(h_hbm.at[srcv.at[pl.ds(nxt * CH, CH)]],
                                     rows_b, sem_b)

        for t in range(MAIN, NCHUNK):
            rows_b, sem_b = bufs[t % NB]
            pltpu.make_async_copy(h_hbm.at[srcv.at[pl.ds(t * CH, CH)]],
                                  rows_b, sem_b).wait()
            pltpu.sync_copy(rows_b, slab.at[dstv.at[t]], add=True)

        plsc.subcore_barrier()

        @pl.when(s < NS - 1)
        def _():
            pltpu.sync_copy(slab.at[pl.ds(s * RPT, RPT)],
                            out_hbm.at[c].at[pl.ds(s * RPT, RPT)])

        @pl.when(s == NS - 1)
        def _():
            pltpu.sync_copy(slab.at[pl.ds((NS - 1) * RPT, RPT_LAST)],
                            out_hbm.at[c].at[pl.ds((NS - 1) * RPT, RPT_LAST)])

    return sc_agg


def _sc_agg(h, src, dst):
    return _build_sc_agg()(h, src, dst)


def _embed_body(x_ref, emb_ref, o_ref):
    xv = x_ref[...]
    onehot = (xv == lax.broadcasted_iota(jnp.int32, (BN, T), 1)).astype(jnp.float32)
    o_ref[...] = jnp.dot(onehot, emb_ref[...], preferred_element_type=jnp.float32)


def _embed(x2, emb):
    return pl.pallas_call(
        _embed_body,
        grid=(N // BN,),
        in_specs=[
            pl.BlockSpec((BN, 1), lambda i: (i, 0)),
            pl.BlockSpec((T, H), lambda i: (0, 0)),
        ],
        out_specs=pl.BlockSpec((BN, H), lambda i: (i, 0)),
        out_shape=jax.ShapeDtypeStruct((N, H), jnp.float32),
    )(x2, emb)


def _mlp_body(h_ref, a0_ref, a1_ref, w1_ref, b1_ref, w2_ref, b2_ref, o_ref):
    z = a0_ref[...] + a1_ref[...] - h_ref[...]
    y = lax.dot_general(z, w1_ref[...], (((1,), (1,)), ((), ())),
                        preferred_element_type=jnp.float32) + b1_ref[...]
    y = jnp.maximum(y, 0.0)
    y = lax.dot_general(y, w2_ref[...], (((1,), (1,)), ((), ())),
                        preferred_element_type=jnp.float32) + b2_ref[...]
    o_ref[...] = jnp.maximum(y, 0.0)


def _mlp(h, a0, a1, w1, b1, w2, b2):
    full = lambda i: (0, 0)
    return pl.pallas_call(
        _mlp_body,
        grid=(N // BN,),
        in_specs=[
            pl.BlockSpec((BN, H), lambda i: (i, 0)),
            pl.BlockSpec((BN, H), lambda i: (i, 0)),
            pl.BlockSpec((BN, H), lambda i: (i, 0)),
            pl.BlockSpec((H, H), full),
            pl.BlockSpec((1, H), full),
            pl.BlockSpec((H, H), full),
            pl.BlockSpec((1, H), full),
        ],
        out_specs=pl.BlockSpec((BN, H), lambda i: (i, 0)),
        out_shape=jax.ShapeDtypeStruct((N, H), jnp.float32),
    )(h, a0, a1, w1, b1, w2, b2)


def _pool_body(b_ref, h_ref, o_ref, cnt_ref):
    i = pl.program_id(0)

    @pl.when(i == 0)
    def _():
        o_ref[...] = jnp.zeros_like(o_ref)
        cnt_ref[...] = jnp.zeros_like(cnt_ref)

    bv = b_ref[...]
    onehot = (bv == lax.broadcasted_iota(jnp.int32, (BN, G), 1)).astype(jnp.float32)
    o_ref[...] += lax.dot_general(onehot, h_ref[...], (((0,), (0,)), ((), ())),
                                  preferred_element_type=jnp.float32)
    cnt_ref[...] += lax.dot_general(onehot, jnp.ones((BN, H), jnp.float32),
                                    (((0,), (0,)), ((), ())),
                                    preferred_element_type=jnp.float32)

    @pl.when(i == pl.num_programs(0) - 1)
    def _():
        o_ref[...] = o_ref[...] / jnp.maximum(cnt_ref[...], 1.0)


def _pool(b2d, h):
    return pl.pallas_call(
        _pool_body,
        grid=(N // BN,),
        in_specs=[
            pl.BlockSpec((BN, 1), lambda i: (i, 0)),
            pl.BlockSpec((BN, H), lambda i: (i, 0)),
        ],
        out_specs=pl.BlockSpec((G, H), lambda i: (0, 0)),
        out_shape=jax.ShapeDtypeStruct((G, H), jnp.float32),
        scratch_shapes=[pltpu.VMEM((G, H), jnp.float32)],
    )(b2d, h)


def kernel(x, edge_index, batch, emb, W1, b1, W2, b2):
    x2 = x.astype(jnp.int32).reshape(N, 1)
    b2d = batch.astype(jnp.int32).reshape(N, 1)
    src = edge_index[0].astype(jnp.int32).reshape(NW, EPT)
    dst = edge_index[1].astype(jnp.int32).reshape(NW, NCHUNK, CH)

    h = _embed(x2, emb)
    for l in range(L):
        agg = _sc_agg(h, src, dst)
        h = _mlp(h, agg[0], agg[1], W1[l], b1[l].reshape(1, H),
                 W2[l], b2[l].reshape(1, H))
    return _pool(b2d, h)


# speedup vs baseline: 11.1177x; 1.0101x over previous
import functools

import jax
import jax.numpy as jnp
from jax import lax
from jax.experimental import pallas as pl
from jax.experimental.pallas import tpu as pltpu
from jax.experimental.pallas import tpu_sc as plsc

N = 10000
E = 320000
H = 128
L = 6
T = 128
G = 64

NC = 2
NS = 16
NW = NC * NS
EPT = E // NW
CH = 80
NCHUNK = EPT // CH
RPT = 640
RPT_LAST = N - (NS - 1) * RPT

BN = 1000

@functools.cache
def _build_sc_agg():
    mesh = plsc.VectorSubcoreMesh(core_axis_name="c", subcore_axis_name="s",
                                  num_cores=NC, num_subcores=NS)

    @functools.partial(
        pl.kernel,
        out_type=jax.ShapeDtypeStruct((NC, N, H), jnp.float32),
        mesh=mesh,
        scratch_types=[
            pltpu.VMEM((EPT,), jnp.int32),
            pltpu.VMEM((NCHUNK, CH), jnp.int32),
            pltpu.VMEM((CH, H), jnp.float32),
            pltpu.VMEM((CH, H), jnp.float32),
            pltpu.VMEM_SHARED((N, H), jnp.float32),
            pltpu.SemaphoreType.DMA,
            pltpu.SemaphoreType.DMA,
        ],
    )
    def sc_agg(h_hbm, src_hbm, dst_hbm, out_hbm, srcv, dstv,
               rows0, rows1, slab, sem0, sem1):
        c = lax.axis_index("c")
        s = lax.axis_index("s")
        wid = s * NC + c
        @pl.when(s < NS - 1)
        def _():
            pltpu.sync_copy(h_hbm.at[pl.ds(s * RPT, RPT)],
                            slab.at[pl.ds(s * RPT, RPT)])

        @pl.when(s == NS - 1)
        def _():
            pltpu.sync_copy(h_hbm.at[pl.ds((NS - 1) * RPT, RPT_LAST)],
                            slab.at[pl.ds((NS - 1) * RPT, RPT_LAST)])

        pltpu.sync_copy(src_hbm.at[wid], srcv)
        pltpu.sync_copy(dst_hbm.at[wid], dstv)

        bufs = [(rows0, sem0), (rows1, sem1)]
        NB = len(bufs)
        for b in range(NB):
            pltpu.async_copy(h_hbm.at[srcv.at[pl.ds(b * CH, CH)]],
                             bufs[b][0], bufs[b][1])
        plsc.subcore_barrier()

        MAIN = ((NCHUNK - NB) // NB) * NB

        @pl.loop(0, MAIN, step=NB)
        def _edges(j):
            for b in range(NB):
                rows_b, sem_b = bufs[b]
                jj = j + b
                pltpu.make_async_copy(h_hbm.at[srcv.at[pl.ds(jj * CH, CH)]],
                                      rows_b, sem_b).wait()
                pltpu.sync_copy(rows_b, slab.at[dstv.at[jj]], add=True)
                pltpu.async_copy(h_hbm.at[srcv.at[pl.ds((jj + NB) * CH, CH)]],
                                 rows_b, sem_b)

        for t in range(MAIN, NCHUNK):
            rows_b, sem_b = bufs[t % NB]
            pltpu.make_async_copy(h_hbm.at[srcv.at[pl.ds(t * CH, CH)]],
                                  rows_b, sem_b).wait()
            pltpu.sync_copy(rows_b, slab.at[dstv.at[t]], add=True)
            nxt = t + NB

            @pl.when(nxt < NCHUNK)
            def _():
                pltpu.async_copy(h_hbm.at[srcv.at[pl.ds(nxt * CH, CH)]],
                                 rows_b, sem_b)

        plsc.subcore_barrier()

        @pl.when(s < NS - 1)
        def _():
            pltpu.sync_copy(slab.at[pl.ds(s * RPT, RPT)],
                            out_hbm.at[c].at[pl.ds(s * RPT, RPT)])

        @pl.when(s == NS - 1)
        def _():
            pltpu.sync_copy(slab.at[pl.ds((NS - 1) * RPT, RPT_LAST)],
                            out_hbm.at[c].at[pl.ds((NS - 1) * RPT, RPT_LAST)])

    return sc_agg


def _sc_agg(h, src, dst):
    return _build_sc_agg()(h, src, dst)


def _embed_body(x_ref, emb_ref, o_ref):
    xv = x_ref[...]
    onehot = (xv == lax.broadcasted_iota(jnp.int32, (BN, T), 1)).astype(jnp.float32)
    o_ref[...] = jnp.dot(onehot, emb_ref[...], preferred_element_type=jnp.float32)


def _embed(x2, emb):
    return pl.pallas_call(
        _embed_body,
        grid=(N // BN,),
        in_specs=[
            pl.BlockSpec((BN, 1), lambda i: (i, 0)),
            pl.BlockSpec((T, H), lambda i: (0, 0)),
        ],
        out_specs=pl.BlockSpec((BN, H), lambda i: (i, 0)),
        out_shape=jax.ShapeDtypeStruct((N, H), jnp.float32),
    )(x2, emb)


def _mlp_body(h_ref, a0_ref, a1_ref, w1_ref, b1_ref, w2_ref, b2_ref, o_ref):
    z = a0_ref[...] + a1_ref[...] - h_ref[...]
    y = lax.dot_general(z, w1_ref[...], (((1,), (1,)), ((), ())),
                        preferred_element_type=jnp.float32) + b1_ref[...]
    y = jnp.maximum(y, 0.0)
    y = lax.dot_general(y, w2_ref[...], (((1,), (1,)), ((), ())),
                        preferred_element_type=jnp.float32) + b2_ref[...]
    o_ref[...] = jnp.maximum(y, 0.0)


def _mlp(h, a0, a1, w1, b1, w2, b2):
    full = lambda i: (0, 0)
    return pl.pallas_call(
        _mlp_body,
        grid=(N // BN,),
        in_specs=[
            pl.BlockSpec((BN, H), lambda i: (i, 0)),
            pl.BlockSpec((BN, H), lambda i: (i, 0)),
            pl.BlockSpec((BN, H), lambda i: (i, 0)),
            pl.BlockSpec((H, H), full),
            pl.BlockSpec((1, H), full),
            pl.BlockSpec((H, H), full),
            pl.BlockSpec((1, H), full),
        ],
        out_specs=pl.BlockSpec((BN, H), lambda i: (i, 0)),
        out_shape=jax.ShapeDtypeStruct((N, H), jnp.float32),
    )(h, a0, a1, w1, b1, w2, b2)


def _mlp_pool_body(b_ref, h_ref, a0_ref, a1_ref, w1_ref, b1_ref, w2_ref,
                   b2_ref, o_ref, cnt_ref):
    i = pl.program_id(0)

    @pl.when(i == 0)
    def _():
        o_ref[...] = jnp.zeros_like(o_ref)
        cnt_ref[...] = jnp.zeros_like(cnt_ref)

    z = a0_ref[...] + a1_ref[...] - h_ref[...]
    y = lax.dot_general(z, w1_ref[...], (((1,), (1,)), ((), ())),
                        preferred_element_type=jnp.float32) + b1_ref[...]
    y = jnp.maximum(y, 0.0)
    y = lax.dot_general(y, w2_ref[...], (((1,), (1,)), ((), ())),
                        preferred_element_type=jnp.float32) + b2_ref[...]
    y = jnp.maximum(y, 0.0)

    bv = b_ref[...]
    onehot = (bv == lax.broadcasted_iota(jnp.int32, (BN, G), 1)).astype(jnp.float32)
    o_ref[...] += lax.dot_general(onehot, y, (((0,), (0,)), ((), ())),
                                  preferred_element_type=jnp.float32)
    cnt_ref[...] += lax.dot_general(onehot, jnp.ones((BN, H), jnp.float32),
                                    (((0,), (0,)), ((), ())),
                                    preferred_element_type=jnp.float32)

    @pl.when(i == pl.num_programs(0) - 1)
    def _():
        o_ref[...] = o_ref[...] / jnp.maximum(cnt_ref[...], 1.0)


def _mlp_pool(b2d, h, a0, a1, w1, b1, w2, b2):
    full = lambda i: (0, 0)
    return pl.pallas_call(
        _mlp_pool_body,
        grid=(N // BN,),
        in_specs=[
            pl.BlockSpec((BN, 1), lambda i: (i, 0)),
            pl.BlockSpec((BN, H), lambda i: (i, 0)),
            pl.BlockSpec((BN, H), lambda i: (i, 0)),
            pl.BlockSpec((BN, H), lambda i: (i, 0)),
            pl.BlockSpec((H, H), full),
            pl.BlockSpec((1, H), full),
            pl.BlockSpec((H, H), full),
            pl.BlockSpec((1, H), full),
        ],
        out_specs=pl.BlockSpec((G, H), lambda i: (0, 0)),
        out_shape=jax.ShapeDtypeStruct((G, H), jnp.float32),
        scratch_shapes=[pltpu.VMEM((G, H), jnp.float32)],
    )(b2d, h, a0, a1, w1, b1, w2, b2)


def kernel(x, edge_index, batch, emb, W1, b1, W2, b2):
    x2 = x.astype(jnp.int32).reshape(N, 1)
    b2d = batch.astype(jnp.int32).reshape(N, 1)
    src = edge_index[0].astype(jnp.int32).reshape(NW, EPT)
    dst = edge_index[1].astype(jnp.int32).reshape(NW, NCHUNK, CH)

    h = _embed(x2, emb)
    for l in range(L - 1):
        agg = _sc_agg(h, src, dst)
        h = _mlp(h, agg[0], agg[1], W1[l], b1[l].reshape(1, H),
                 W2[l], b2[l].reshape(1, H))
    agg = _sc_agg(h, src, dst)
    return _mlp_pool(b2d, h, agg[0], agg[1], W1[L - 1], b1[L - 1].reshape(1, H),
                     W2[L - 1], b2[L - 1].reshape(1, H))
